# Initial kernel scaffold; baseline (speedup 1.0000x reference)
import jax
import jax.numpy as jnp
from jax.experimental import pallas as pl


def kernel(x, edge_index, W1, att_src1, att_dst1, bias1, W2, att_src2, att_dst2, bias2):
    raise NotImplementedError("write your pallas kernel here")


# speedup vs baseline: 31.2757x; 31.2757x over previous
import functools
import jax
import jax.numpy as jnp
from jax import lax
from jax.experimental import pallas as pl
from jax.experimental.pallas import tpu as pltpu
from jax.experimental.pallas import tpu_sc as plsc

N = 10000
E = 320000
NFEAT = 128
NHID = 16
HEADS = 8
NCLASS = 40

NC = 2
NS = 16
NW = NC * NS
CHUNK = 128
CPW = -(-((E + NW * CHUNK - 1) // (NW * CHUNK)) // 8) * 8
E_PAD = NW * CPW * CHUNK
PAD_DST = N + 8
N_ACC = ((PAD_DST + 127) // 128) * 128

ROW1 = 80
ROW2 = 64


def _headsel(nlanes, heads, width, dtype):
  r = lax.broadcasted_iota(jnp.int32, (nlanes, heads), 0) // width
  c = lax.broadcasted_iota(jnp.int32, (nlanes, heads), 1)
  return (r == c).astype(dtype)


def _tc_a(x_ref, w1_ref, as1_ref, ad1_ref, t1a_ref, t1b_ref, adt1_ref,
          mvec_ref):
  h = jnp.dot(x_ref[...], w1_ref[...], preferred_element_type=jnp.float32)
  sel = _headsel(HEADS * NHID, HEADS, NHID, jnp.float32)
  a_src = jnp.dot(h * as1_ref[...], sel, preferred_element_type=jnp.float32)
  a_dst = jnp.dot(h * ad1_ref[...], sel, preferred_element_type=jnp.float32)
  m8 = jnp.maximum(
      0.0,
      jnp.max(a_src, axis=0, keepdims=True)
      + jnp.max(a_dst, axis=0, keepdims=True))
  mvec_ref[...] = jnp.concatenate([m8, m8], axis=1)
  z8 = jnp.zeros((N, 8), jnp.float32)
  t1a_ref[...] = jnp.concatenate([a_src, z8, h[:, 0:64]], axis=1)
  t1b_ref[...] = jnp.concatenate([a_src, z8, h[:, 64:128]], axis=1)
  adt = jnp.concatenate([a_dst, z8], axis=1)
  adt1_ref[...] = jnp.concatenate(
      [adt, jnp.zeros((N_ACC - N, 16), jnp.float32)], axis=0)


def _make_sc_kernel(roww, nmsg, head_off):
  mesh = plsc.VectorSubcoreMesh(core_axis_name="c", subcore_axis_name="s")

  @functools.partial(
      pl.kernel,
      out_type=jax.ShapeDtypeStruct((NC, N_ACC, roww), jnp.float32),
      mesh=mesh,
      compiler_params=pltpu.CompilerParams(use_tc_tiling_on_sc=False),
      scratch_types=[
          pltpu.VMEM((CPW, CHUNK), jnp.int32),
          pltpu.VMEM((CPW, CHUNK), jnp.int32),
          pltpu.VMEM((CHUNK, roww), jnp.float32),
          pltpu.VMEM((CHUNK, 16), jnp.float32),
          pltpu.VMEM((CHUNK, roww), jnp.float32),
          pltpu.VMEM((16,), jnp.float32),
          pltpu.VMEM_SHARED((N_ACC, roww), jnp.float32),
          pltpu.SemaphoreType.DMA,
          pltpu.SemaphoreType.DMA,
      ],
  )
  def sc_kernel(src_hbm, dst_hbm, t_hbm, ad_hbm, mvec_hbm, zrows_hbm,
                out_hbm, src_v, dst_v, g_v, d_v, o_v, m_v, acc_sh,
                sem_g, sem_d):
    cid = lax.axis_index("c")
    sid = lax.axis_index("s")
    wid = cid * NS + sid
    base = wid * CPW

    pltpu.sync_copy(src_hbm.at[pl.ds(base, CPW)], src_v)
    pltpu.sync_copy(dst_hbm.at[pl.ds(base, CPW)], dst_v)
    pltpu.sync_copy(mvec_hbm, m_v)
    mv = m_v[...]

    nblk = N_ACC // CHUNK
    def zero_body(i, carry):
      @pl.when(i % NS == sid)
      def _():
        pltpu.sync_copy(zrows_hbm, acc_sh.at[pl.ds(i * CHUNK, CHUNK)])
      return carry
    lax.fori_loop(0, nblk, zero_body, 0)
    plsc.subcore_barrier()

    def chunk_body(c, carry):
      cp_g = pltpu.async_copy(t_hbm.at[src_v.at[c]], g_v, sem_g)
      cp_d = pltpu.async_copy(ad_hbm.at[dst_v.at[c]], d_v, sem_d)
      cp_g.wait()
      cp_d.wait()

      def edge_body(e, carry2):
        ga = g_v[e, pl.ds(0, 16)]
        da = d_v[e, pl.ds(0, 16)]
        al = ga + da
        lr = jnp.where(al >= 0.0, al, 0.2 * al)
        w = jnp.exp(lr - mv)
        o_v[e, pl.ds(0, 16)] = w
        for j in range(nmsg):
          hv = g_v[e, pl.ds(16 + 16 * j, 16)]
          wj = w[0] if head_off is None else w[head_off + j]
          o_v[e, pl.ds(16 + 16 * j, 16)] = wj * hv
        return carry2
      lax.fori_loop(0, CHUNK, edge_body, 0)

      pltpu.sync_copy(o_v, acc_sh.at[dst_v.at[c]], add=True)
      return carry
    lax.fori_loop(0, CPW, chunk_body, 0)

    plsc.subcore_barrier()
    rows = N_ACC // NS
    pltpu.sync_copy(acc_sh.at[pl.ds(sid * rows, rows)],
                    out_hbm.at[cid, pl.ds(sid * rows, rows)])

  return sc_kernel


BR = 1024


def _tc_b(acca_ref, accb_ref, t1a_ref, t1b_ref, adt1_ref, mvec_ref, w2_ref,
          as2_ref, ad2_ref, b1_ref, t2_ref, adt2_ref):
  a_s = t1a_ref[:, 0:8]
  h = jnp.concatenate([t1a_ref[:, 16:80], t1b_ref[:, 16:80]], axis=1)
  a_d = adt1_ref[:, 0:8]
  m8 = mvec_ref[0:1, 0:8]
  al = a_s + a_d
  lr = jnp.where(al >= 0.0, al, 0.2 * al)
  wself = jnp.exp(lr - m8)
  selT = _headsel(HEADS * NHID, HEADS, NHID, jnp.float32).T
  wx = jnp.dot(wself, selT, preferred_element_type=jnp.float32)
  num = jnp.concatenate(
      [acca_ref[0, :, 16:80] + acca_ref[1, :, 16:80],
       accb_ref[0, :, 16:80] + accb_ref[1, :, 16:80]], axis=1)
  num = num + wx * h
  den8 = acca_ref[0, :, 0:8] + acca_ref[1, :, 0:8] + wself
  denx = jnp.dot(den8, selT, preferred_element_type=jnp.float32)
  v = num / (denx + 1e-16) + b1_ref[...]
  h1p = jnp.where(v > 0.0, v, jnp.exp(v) - 1.0)
  h2 = jnp.dot(h1p, w2_ref[...], preferred_element_type=jnp.float32)
  a_s2 = jnp.sum(h2 * as2_ref[...], axis=1, keepdims=True)
  a_d2 = jnp.sum(h2 * ad2_ref[...], axis=1, keepdims=True)
  z15 = jnp.zeros((BR, 15), jnp.float32)
  t2_ref[...] = jnp.concatenate(
      [a_s2, z15, h2, jnp.zeros((BR, 8), jnp.float32)], axis=1)
  row = (pl.program_id(0) * BR
         + lax.broadcasted_iota(jnp.int32, (BR, 1), 0))
  a_d2 = jnp.where(row < N, a_d2, 0.0)
  adt2_ref[...] = jnp.concatenate([a_d2, z15], axis=1)


def _tc_b2(t2_ref, adt2_ref, m2vec_ref):
  a_s2 = t2_ref[:, 0:1]
  a_d2 = adt2_ref[0:N, 0:1]
  m2 = jnp.maximum(
      0.0,
      jnp.max(a_s2, axis=0, keepdims=True)
      + jnp.max(a_d2, axis=0, keepdims=True))
  m2vec_ref[...] = jax.lax.broadcast_in_dim(m2, (1, 16), (0, 1))


def _tc_c(acc_ref, t2_ref, adt2_ref, m2vec_ref, b2_ref, out_ref):
  a_s2 = t2_ref[:, 0:1]
  h2 = t2_ref[:, 16:56]
  a_d2 = adt2_ref[0:N, 0:1]
  m2 = m2vec_ref[0:1, 0:1]
  al = a_s2 + a_d2
  lr = jnp.where(al >= 0.0, al, 0.2 * al)
  ws = jnp.exp(lr - m2)
  num = acc_ref[0, 0:N, 16:56] + acc_ref[1, 0:N, 16:56] + ws * h2
  den = acc_ref[0, 0:N, 0:1] + acc_ref[1, 0:N, 0:1] + ws
  y = num / (den + 1e-16) + b2_ref[...]
  mrow = jnp.max(y, axis=1, keepdims=True)
  s = y - mrow
  out_ref[...] = s - jnp.log(jnp.sum(jnp.exp(s), axis=1, keepdims=True))


def kernel(x, edge_index, W1, att_src1, att_dst1, bias1,
           W2, att_src2, att_dst2, bias2):
  as1f = att_src1.reshape(1, HEADS * NHID)
  ad1f = att_dst1.reshape(1, HEADS * NHID)
  as2f = att_src2.reshape(1, NCLASS)
  ad2f = att_dst2.reshape(1, NCLASS)
  b1 = bias1.reshape(1, HEADS * NHID)
  b2 = bias2.reshape(1, NCLASS)

  npad = E_PAD - E
  src_pad = jnp.concatenate(
      [edge_index[0], jnp.zeros((npad,), jnp.int32)]).reshape(-1, CHUNK)
  dst_pad = jnp.concatenate(
      [edge_index[1], jnp.full((npad,), PAD_DST, jnp.int32)]).reshape(-1, CHUNK)

  t1a, t1b, adt1, mvec = pl.pallas_call(
      _tc_a,
      out_shape=[
          jax.ShapeDtypeStruct((N, ROW1), jnp.float32),
          jax.ShapeDtypeStruct((N, ROW1), jnp.float32),
          jax.ShapeDtypeStruct((N_ACC, 16), jnp.float32),
          jax.ShapeDtypeStruct((1, 16), jnp.float32),
      ],
  )(x, W1, as1f, ad1f)

  zrows1 = jnp.zeros((CHUNK, ROW1), jnp.float32)
  mvec_flat = mvec.reshape(16)
  sc1a = _make_sc_kernel(ROW1, 4, 0)
  sc1b = _make_sc_kernel(ROW1, 4, 4)
  acc1a = sc1a(src_pad, dst_pad, t1a, adt1, mvec_flat, zrows1)
  acc1b = sc1b(src_pad, dst_pad, t1b, adt1, mvec_flat, zrows1)

  nb = -(-N_ACC // BR)
  t2, adt2 = pl.pallas_call(
      _tc_b,
      grid=(nb,),
      in_specs=[
          pl.BlockSpec((2, BR, ROW1), lambda i: (0, i, 0)),
          pl.BlockSpec((2, BR, ROW1), lambda i: (0, i, 0)),
          pl.BlockSpec((BR, ROW1), lambda i: (i, 0)),
          pl.BlockSpec((BR, ROW1), lambda i: (i, 0)),
          pl.BlockSpec((BR, 16), lambda i: (i, 0)),
          pl.BlockSpec((1, 16), lambda i: (0, 0)),
          pl.BlockSpec((NFEAT, NCLASS), lambda i: (0, 0)),
          pl.BlockSpec((1, NCLASS), lambda i: (0, 0)),
          pl.BlockSpec((1, NCLASS), lambda i: (0, 0)),
          pl.BlockSpec((1, NFEAT), lambda i: (0, 0)),
      ],
      out_specs=[
          pl.BlockSpec((BR, ROW2), lambda i: (i, 0)),
          pl.BlockSpec((BR, 16), lambda i: (i, 0)),
      ],
      out_shape=[
          jax.ShapeDtypeStruct((N, ROW2), jnp.float32),
          jax.ShapeDtypeStruct((N_ACC, 16), jnp.float32),
      ],
  )(acc1a, acc1b, t1a, t1b, adt1, mvec, W2, as2f, ad2f, b1)

  m2vec = pl.pallas_call(
      _tc_b2,
      out_shape=jax.ShapeDtypeStruct((1, 16), jnp.float32),
  )(t2, adt2)

  zrows2 = jnp.zeros((CHUNK, ROW2), jnp.float32)
  sc2 = _make_sc_kernel(ROW2, 3, None)
  acc2 = sc2(src_pad, dst_pad, t2, adt2, m2vec.reshape(16), zrows2)

  out = pl.pallas_call(
      _tc_c,
      out_shape=jax.ShapeDtypeStruct((N, NCLASS), jnp.float32),
  )(acc2, t2, adt2, m2vec, b2)
  return out


# speedup vs baseline: 41.8051x; 1.3367x over previous
import functools
import jax
import jax.numpy as jnp
from jax import lax
from jax.experimental import pallas as pl
from jax.experimental.pallas import tpu as pltpu
from jax.experimental.pallas import tpu_sc as plsc

N = 10000
E = 320000
NFEAT = 128
NHID = 16
HEADS = 8
NCLASS = 40

NC = 2
NS = 16
NW = NC * NS
CHUNK = 128
CPW32 = -(-((E + NW * CHUNK - 1) // (NW * CHUNK)) // 8) * 8
E_PAD = NW * CPW32 * CHUNK
NCHUNKS = E_PAD // CHUNK
PAD_DST = N + 8
N_ACC = ((PAD_DST + 127) // 128) * 128

ROW1 = 80
ROW2 = 64


def _headsel(nlanes, heads, width, dtype):
  r = lax.broadcasted_iota(jnp.int32, (nlanes, heads), 0) // width
  c = lax.broadcasted_iota(jnp.int32, (nlanes, heads), 1)
  return (r == c).astype(dtype)


def _tc_a(x_ref, w1_ref, as1_ref, ad1_ref, t1a_ref, t1b_ref, adt1_ref,
          mvec_ref):
  h = jnp.dot(x_ref[...], w1_ref[...], preferred_element_type=jnp.float32)
  sel = _headsel(HEADS * NHID, HEADS, NHID, jnp.float32)
  a_src = jnp.dot(h * as1_ref[...], sel, preferred_element_type=jnp.float32)
  a_dst = jnp.dot(h * ad1_ref[...], sel, preferred_element_type=jnp.float32)
  m8 = jnp.maximum(
      0.0,
      jnp.max(a_src, axis=0, keepdims=True)
      + jnp.max(a_dst, axis=0, keepdims=True))
  mvec_ref[...] = jnp.concatenate([m8, m8], axis=1)
  z8 = jnp.zeros((N, 8), jnp.float32)
  t1a_ref[...] = jnp.concatenate([a_src, z8, h[:, 0:64]], axis=1)
  t1b_ref[...] = jnp.concatenate([a_src, z8, h[:, 64:128]], axis=1)
  adt = jnp.concatenate([a_dst, z8], axis=1)
  adt1_ref[...] = jnp.concatenate(
      [adt, jnp.zeros((N_ACC - N, 16), jnp.float32)], axis=0)


def _make_sc_kernel(roww, nmsg, head_offs, core_split):
  cpw = NCHUNKS // NS if core_split else NCHUNKS // NW

  mesh = plsc.VectorSubcoreMesh(core_axis_name="c", subcore_axis_name="s")

  @functools.partial(
      pl.kernel,
      out_type=jax.ShapeDtypeStruct((NC, N_ACC, roww), jnp.float32),
      mesh=mesh,
      compiler_params=pltpu.CompilerParams(use_tc_tiling_on_sc=False),
      scratch_types=[
          pltpu.VMEM((4, CHUNK), jnp.int32),
          pltpu.VMEM((4, CHUNK), jnp.int32),
          pltpu.VMEM((CHUNK, roww), jnp.float32),
          pltpu.VMEM((CHUNK, roww), jnp.float32),
          pltpu.VMEM((CHUNK, 16), jnp.float32),
          pltpu.VMEM((CHUNK, 16), jnp.float32),
          pltpu.VMEM((CHUNK, roww), jnp.float32),
          pltpu.VMEM((CHUNK, roww), jnp.float32),
          pltpu.VMEM((16,), jnp.float32),
          pltpu.VMEM_SHARED((N_ACC, roww), jnp.float32),
          pltpu.SemaphoreType.DMA,
          pltpu.SemaphoreType.DMA,
          pltpu.SemaphoreType.DMA,
          pltpu.SemaphoreType.DMA,
          pltpu.SemaphoreType.DMA,
          pltpu.SemaphoreType.DMA,
          pltpu.SemaphoreType.DMA,
          pltpu.SemaphoreType.DMA,
      ],
  )
  def sc_kernel(src_hbm, dst_hbm, ta_hbm, tb_hbm, ad_hbm, mvec_hbm,
                zrows_hbm, out_hbm, srow, drow, g0, g1, d0, d1, o0, o1,
                m_v, acc_sh, si0, si1, si2, si3, sg0, sg1, ss0, ss1):
    cid = lax.axis_index("c")
    sid = lax.axis_index("s")
    base = sid * cpw if core_split else (cid * NS + sid) * cpw

    pltpu.sync_copy(mvec_hbm, m_v)
    mv = m_v[...]

    nblk = N_ACC // CHUNK
    def zero_body(i, carry):
      @pl.when(i % NS == sid)
      def _():
        pltpu.sync_copy(zrows_hbm, acc_sh.at[pl.ds(i * CHUNK, CHUNK)])
      return carry
    lax.fori_loop(0, nblk, zero_body, 0)
    plsc.subcore_barrier()

    g_slots = (g0, g1)
    d_slots = (d0, d1)
    o_slots = (o0, o1)
    si = (si0, si1, si2, si3)
    sg = (sg0, sg1)
    ss = (ss0, ss1)

    def run_pass(tbl, hoff):
      def issue_idx(s4, c):
        pltpu.async_copy(src_hbm.at[base + c], srow.at[s4], si[s4])
        pltpu.async_copy(dst_hbm.at[base + c], drow.at[s4], si[s4])

      def wait_idx(s4, c):
        pltpu.make_async_copy(src_hbm.at[base + c], srow.at[s4],
                              si[s4]).wait()
        pltpu.make_async_copy(dst_hbm.at[base + c], drow.at[s4],
                              si[s4]).wait()

      def issue_gather(s2, s4):
        pltpu.async_copy(tbl.at[srow.at[s4]], g_slots[s2], sg[s2])
        pltpu.async_copy(ad_hbm.at[drow.at[s4]], d_slots[s2], sg[s2])

      def wait_gather(s2, s4):
        pltpu.make_async_copy(tbl.at[srow.at[s4]], g_slots[s2],
                              sg[s2]).wait()
        pltpu.make_async_copy(ad_hbm.at[drow.at[s4]], d_slots[s2],
                              sg[s2]).wait()

      def compute(s2):
        g_v = g_slots[s2]
        d_v = d_slots[s2]
        o_v = o_slots[s2]

        def edge_body(e, carry2):
          ga = g_v[e, pl.ds(0, 16)]
          da = d_v[e, pl.ds(0, 16)]
          al = ga + da
          lr = jnp.maximum(al, 0.2 * al)
          w = jnp.exp(lr - mv)
          o_v[e, pl.ds(0, 16)] = w
          for j in range(nmsg):
            hv = g_v[e, pl.ds(16 + 16 * j, 16)]
            wj = w[0] if head_offs is None else w[hoff + j]
            o_v[e, pl.ds(16 + 16 * j, 16)] = wj * hv
          return carry2
        lax.fori_loop(0, CHUNK, edge_body, 0, unroll=4)

      issue_idx(0, 0)
      issue_idx(1, 1)
      wait_idx(0, 0)
      issue_gather(0, 0)

      def outer(i, carry):
        c0 = 4 * i
        for b in range(4):
          c = c0 + b
          s2 = b % 2
          s4 = b
          wait_gather(s2, s4)
          @pl.when(c >= 2)
          def _():
            pltpu.make_async_copy(
                o_slots[s2], acc_sh.at[drow.at[(b + 2) % 4]],
                ss[s2]).wait()
          @pl.when(c + 2 < cpw)
          def _():
            issue_idx((b + 2) % 4, c + 2)
          @pl.when(c + 1 < cpw)
          def _():
            wait_idx((b + 1) % 4, c + 1)
            issue_gather(1 - s2, (b + 1) % 4)
          compute(s2)
          pltpu.async_copy(o_slots[s2], acc_sh.at[drow.at[s4]], ss[s2],
                           add=True)
        return carry
      lax.fori_loop(0, cpw // 4, outer, 0)

      for b in range(2):
        c = cpw - 2 + b
        pltpu.make_async_copy(
            o_slots[c % 2], acc_sh.at[drow.at[c % 4]], ss[c % 2]).wait()

    if core_split:
      @pl.when(cid == 0)
      def _():
        run_pass(ta_hbm, None if head_offs is None else head_offs[0])
      @pl.when(cid == 1)
      def _():
        run_pass(tb_hbm, None if head_offs is None else head_offs[1])
    else:
      run_pass(ta_hbm, None if head_offs is None else head_offs[0])

    plsc.subcore_barrier()
    rows = N_ACC // NS
    pltpu.sync_copy(acc_sh.at[pl.ds(sid * rows, rows)],
                    out_hbm.at[cid, pl.ds(sid * rows, rows)])

  return sc_kernel


BR = 1024


def _tc_b(acc_ref, t1a_ref, t1b_ref, adt1_ref, mvec_ref, w2_ref,
          as2_ref, ad2_ref, b1_ref, t2_ref, adt2_ref):
  a_s = t1a_ref[:, 0:8]
  h = jnp.concatenate([t1a_ref[:, 16:80], t1b_ref[:, 16:80]], axis=1)
  a_d = adt1_ref[:, 0:8]
  m8 = mvec_ref[0:1, 0:8]
  al = a_s + a_d
  lr = jnp.maximum(al, 0.2 * al)
  wself = jnp.exp(lr - m8)
  selT = _headsel(HEADS * NHID, HEADS, NHID, jnp.float32).T
  wx = jnp.dot(wself, selT, preferred_element_type=jnp.float32)
  num = jnp.concatenate(
      [acc_ref[0, :, 16:80], acc_ref[1, :, 16:80]], axis=1)
  num = num + wx * h
  den8 = acc_ref[0, :, 0:8] + wself
  denx = jnp.dot(den8, selT, preferred_element_type=jnp.float32)
  v = num / (denx + 1e-16) + b1_ref[...]
  h1p = jnp.where(v > 0.0, v, jnp.exp(v) - 1.0)
  h2 = jnp.dot(h1p, w2_ref[...], preferred_element_type=jnp.float32)
  a_s2 = jnp.sum(h2 * as2_ref[...], axis=1, keepdims=True)
  a_d2 = jnp.sum(h2 * ad2_ref[...], axis=1, keepdims=True)
  z15 = jnp.zeros((BR, 15), jnp.float32)
  t2_ref[...] = jnp.concatenate(
      [a_s2, z15, h2, jnp.zeros((BR, 8), jnp.float32)], axis=1)
  row = (pl.program_id(0) * BR
         + lax.broadcasted_iota(jnp.int32, (BR, 1), 0))
  a_d2 = jnp.where(row < N, a_d2, 0.0)
  adt2_ref[...] = jnp.concatenate([a_d2, z15], axis=1)


def _tc_b2(t2_ref, adt2_ref, m2vec_ref):
  a_s2 = t2_ref[:, 0:1]
  a_d2 = adt2_ref[0:N, 0:1]
  m2 = jnp.maximum(
      0.0,
      jnp.max(a_s2, axis=0, keepdims=True)
      + jnp.max(a_d2, axis=0, keepdims=True))
  m2vec_ref[...] = jax.lax.broadcast_in_dim(m2, (1, 16), (0, 1))


def _tc_c(acc_ref, t2_ref, adt2_ref, m2vec_ref, b2_ref, out_ref):
  a_s2 = t2_ref[:, 0:1]
  h2 = t2_ref[:, 16:56]
  a_d2 = adt2_ref[0:N, 0:1]
  m2 = m2vec_ref[0:1, 0:1]
  al = a_s2 + a_d2
  lr = jnp.maximum(al, 0.2 * al)
  ws = jnp.exp(lr - m2)
  num = acc_ref[0, 0:N, 16:56] + acc_ref[1, 0:N, 16:56] + ws * h2
  den = acc_ref[0, 0:N, 0:1] + acc_ref[1, 0:N, 0:1] + ws
  y = num / (den + 1e-16) + b2_ref[...]
  mrow = jnp.max(y, axis=1, keepdims=True)
  s = y - mrow
  out_ref[...] = s - jnp.log(jnp.sum(jnp.exp(s), axis=1, keepdims=True))


def kernel(x, edge_index, W1, att_src1, att_dst1, bias1,
           W2, att_src2, att_dst2, bias2):
  as1f = att_src1.reshape(1, HEADS * NHID)
  ad1f = att_dst1.reshape(1, HEADS * NHID)
  as2f = att_src2.reshape(1, NCLASS)
  ad2f = att_dst2.reshape(1, NCLASS)
  b1 = bias1.reshape(1, HEADS * NHID)
  b2 = bias2.reshape(1, NCLASS)

  npad = E_PAD - E
  src_pad = jnp.concatenate(
      [edge_index[0], jnp.zeros((npad,), jnp.int32)]).reshape(-1, CHUNK)
  dst_pad = jnp.concatenate(
      [edge_index[1], jnp.full((npad,), PAD_DST, jnp.int32)]).reshape(-1, CHUNK)

  t1a, t1b, adt1, mvec = pl.pallas_call(
      _tc_a,
      out_shape=[
          jax.ShapeDtypeStruct((N, ROW1), jnp.float32),
          jax.ShapeDtypeStruct((N, ROW1), jnp.float32),
          jax.ShapeDtypeStruct((N_ACC, 16), jnp.float32),
          jax.ShapeDtypeStruct((1, 16), jnp.float32),
      ],
  )(x, W1, as1f, ad1f)

  zrows1 = jnp.zeros((CHUNK, ROW1), jnp.float32)
  mvec_flat = mvec.reshape(16)
  sc1 = _make_sc_kernel(ROW1, 4, (0, 4), True)
  acc1 = sc1(src_pad, dst_pad, t1a, t1b, adt1, mvec_flat, zrows1)

  nb = -(-N_ACC // BR)
  t2, adt2 = pl.pallas_call(
      _tc_b,
      grid=(nb,),
      in_specs=[
          pl.BlockSpec((2, BR, ROW1), lambda i: (0, i, 0)),
          pl.BlockSpec((BR, ROW1), lambda i: (i, 0)),
          pl.BlockSpec((BR, ROW1), lambda i: (i, 0)),
          pl.BlockSpec((BR, 16), lambda i: (i, 0)),
          pl.BlockSpec((1, 16), lambda i: (0, 0)),
          pl.BlockSpec((NFEAT, NCLASS), lambda i: (0, 0)),
          pl.BlockSpec((1, NCLASS), lambda i: (0, 0)),
          pl.BlockSpec((1, NCLASS), lambda i: (0, 0)),
          pl.BlockSpec((1, NFEAT), lambda i: (0, 0)),
      ],
      out_specs=[
          pl.BlockSpec((BR, ROW2), lambda i: (i, 0)),
          pl.BlockSpec((BR, 16), lambda i: (i, 0)),
      ],
      out_shape=[
          jax.ShapeDtypeStruct((N, ROW2), jnp.float32),
          jax.ShapeDtypeStruct((N_ACC, 16), jnp.float32),
      ],
  )(acc1, t1a, t1b, adt1, mvec, W2, as2f, ad2f, b1)

  m2vec = pl.pallas_call(
      _tc_b2,
      out_shape=jax.ShapeDtypeStruct((1, 16), jnp.float32),
  )(t2, adt2)

  zrows2 = jnp.zeros((CHUNK, ROW2), jnp.float32)
  sc2 = _make_sc_kernel(ROW2, 3, None, False)
  acc2 = sc2(src_pad, dst_pad, t2, t2, adt2, m2vec.reshape(16), zrows2)

  out = pl.pallas_call(
      _tc_c,
      out_shape=jax.ShapeDtypeStruct((N, NCLASS), jnp.float32),
  )(acc2, t2, adt2, m2vec, b2)
  return out
